# baseline (speedup 1.0000x reference)
import functools

import jax
import jax.numpy as jnp
from jax import lax
from jax.experimental import pallas as pl
from jax.experimental.pallas import tpu as pltpu
from jax.experimental.pallas import tpu_sc as plsc

N = 10000
E = 160000
D = 256
ROW_BLK = 1000

NW = 32
RPT = 320
NPAD = NW * RPT
CAP = E + 128
GB = 128


def _widx():
    info = plsc.get_sparse_core_info()
    return lax.axis_index("s") * info.num_cores + lax.axis_index("c")


def _build_routing(src, dst):
    w = dst // RPT
    order = jnp.argsort(w)
    ssrc = src[order]
    sdst = dst[order]
    sw = w[order]
    wids = jnp.arange(NW, dtype=jnp.int32)
    starts = jnp.searchsorted(sw, wids, side="left")
    ends = jnp.searchsorted(sw, wids, side="right")
    counts = (ends - starts).astype(jnp.int32)
    pos = jnp.arange(E, dtype=jnp.int32) - starts[sw].astype(jnp.int32)
    slot = sw * CAP + pos
    lsrc = jnp.zeros((NW * CAP,), jnp.int32).at[slot].set(ssrc)
    ldst = jnp.full((NW * CAP,), RPT, jnp.int32).at[slot].set(
        (sdst - sw * RPT).astype(jnp.int32))
    cpad = ((counts + GB - 1) // GB) * GB
    cnts = jnp.repeat(cpad, 16).astype(jnp.int32)
    return lsrc, ldst, cnts


def _segmax_body(m_hbm, lsrc_hbm, ldst_hbm, cnt_hbm, out_hbm,
                 acc, idxv, dstv, rows, cntv, sem):
    wid = _widx()
    base = wid * CAP

    zrow = jnp.zeros((16,), jnp.float32)

    def zero_body(r, _):
        for c in range(D // 16):
            acc[r, pl.ds(c * 16, 16)] = zrow
        return 0

    lax.fori_loop(0, RPT + 1, zero_body, 0)

    pltpu.sync_copy(cnt_hbm.at[pl.ds(pl.multiple_of(wid * 16, 8), 16)], cntv)
    cnt = cntv[...][0]

    def batch_body(b, _):
        @pl.when(b * GB < cnt)
        def _():
            g = pl.multiple_of(base + b * GB, 8)
            pltpu.sync_copy(lsrc_hbm.at[pl.ds(g, GB)], idxv)
            pltpu.sync_copy(ldst_hbm.at[pl.ds(g, GB)], dstv)
            pltpu.async_copy(m_hbm.at[idxv], rows, sem).wait()

            def grp_body(t, _):
                dvec = dstv[pl.ds(t * 16, 16)]
                for k in range(16):
                    dl = dvec[k]
                    j = t * 16 + k

                    def col_body(c, _):
                        sl = pl.ds(pl.multiple_of(c * 16, 16), 16)
                        acc[dl, sl] = jnp.maximum(acc[dl, sl], rows[j, sl])
                        return 0

                    lax.fori_loop(0, D // 16, col_body, 0)
                return 0

            lax.fori_loop(0, GB // 16, grp_body, 0)

        return 0

    lax.fori_loop(0, CAP // GB, batch_body, 0)
    pltpu.sync_copy(acc.at[pl.ds(0, RPT)], out_hbm.at[pl.ds(wid * RPT, RPT)])


def _segmax_sc(m, lsrc, ldst, cnts):
    mesh = plsc.VectorSubcoreMesh(core_axis_name="c", subcore_axis_name="s")
    f = pl.kernel(
        _segmax_body,
        out_type=jax.ShapeDtypeStruct((NPAD, D), jnp.float32),
        mesh=mesh,
        scratch_types=(
            pltpu.VMEM((RPT + 1, D), jnp.float32),
            pltpu.VMEM((GB,), jnp.int32),
            pltpu.VMEM((GB,), jnp.int32),
            pltpu.VMEM((GB, D), jnp.float32),
            pltpu.VMEM((16,), jnp.int32),
            pltpu.SemaphoreType.DMA,
        ),
    )
    return f(m, lsrc, ldst, cnts)


def _pool_mm_body(h_ref, w_ref, b_ref, o_ref):
    o_ref[...] = jnp.maximum(
        jnp.dot(h_ref[...], w_ref[...], preferred_element_type=jnp.float32)
        + b_ref[...], 0.0)


def _pool_mm(h, w, b):
    return pl.pallas_call(
        _pool_mm_body,
        grid=(N // ROW_BLK,),
        in_specs=[
            pl.BlockSpec((ROW_BLK, D), lambda i: (i, 0)),
            pl.BlockSpec((D, D), lambda i: (0, 0)),
            pl.BlockSpec((D,), lambda i: (0,)),
        ],
        out_specs=pl.BlockSpec((ROW_BLK, D), lambda i: (i, 0)),
        out_shape=jax.ShapeDtypeStruct((N, D), jnp.float32),
    )(h, w, b)


def _out_mm_body(h_ref, hn_ref, ws_ref, wn_ref, b_ref, o_ref, *, act):
    r = (jnp.dot(h_ref[...], ws_ref[...], preferred_element_type=jnp.float32)
         + jnp.dot(hn_ref[...], wn_ref[...], preferred_element_type=jnp.float32)
         + b_ref[...])
    o_ref[...] = jnp.tanh(r) if act else r


def _out_mm(h, hn, ws, wn, b, act):
    return pl.pallas_call(
        functools.partial(_out_mm_body, act=act),
        grid=(N // ROW_BLK,),
        in_specs=[
            pl.BlockSpec((ROW_BLK, D), lambda i: (i, 0)),
            pl.BlockSpec((ROW_BLK, D), lambda i: (i, 0)),
            pl.BlockSpec((D, D), lambda i: (0, 0)),
            pl.BlockSpec((D, D), lambda i: (0, 0)),
            pl.BlockSpec((D,), lambda i: (0,)),
        ],
        out_specs=pl.BlockSpec((ROW_BLK, D), lambda i: (i, 0)),
        out_shape=jax.ShapeDtypeStruct((N, D), jnp.float32),
    )(h, hn, ws, wn, b)


def kernel(x, edge_index,
           W_pool1, b_pool1, W_self1, W_neigh1, b1,
           W_pool2, b_pool2, W_self2, W_neigh2, b2,
           W_pool3, b_pool3, W_self3, W_neigh3, b3):
    src = edge_index[0]
    dst = edge_index[1]
    lsrc, ldst, cnts = _build_routing(src, dst)
    params = [
        (W_pool1, b_pool1, W_self1, W_neigh1, b1, True),
        (W_pool2, b_pool2, W_self2, W_neigh2, b2, True),
        (W_pool3, b_pool3, W_self3, W_neigh3, b3, False),
    ]
    h = x
    for wp, bp, ws, wn, b, act in params:
        m = _pool_mm(h, wp, bp)
        hn = _segmax_sc(m, lsrc, ldst, cnts)[:N]
        h = _out_mm(h, hn, ws, wn, b, act)
    return h
